# baseline (speedup 1.0000x reference)
import functools
import math

import jax
import jax.numpy as jnp
from jax import lax
from jax.experimental import pallas as pl
from jax.experimental.pallas import tpu as pltpu
from jax.experimental.pallas import tpu_sc as plsc

_D_MODEL = 100000
_D = 64
_SCALE = math.sqrt(_D_MODEL)
_NC, _NS = 2, 16
_NW = _NC * _NS
_NI, _NJ = 4096, 200
_B = _NI * _NJ
_IB = _NI // _NW
_B_PER_W = _IB * _NJ

_mesh = plsc.VectorSubcoreMesh(core_axis_name="c", subcore_axis_name="s")


@functools.partial(
    pl.kernel,
    out_type=jax.ShapeDtypeStruct((_NJ, 8, _NW, 1024), jnp.float32),
    mesh=_mesh,
    scratch_types=[
        pltpu.VMEM((_B_PER_W,), jnp.int32),
        pltpu.VMEM((_IB,), jnp.int32),
        pltpu.VMEM((_IB,), jnp.int32),
        pltpu.VMEM((_IB, _D), jnp.float32),
        pltpu.VMEM((_IB, _D), jnp.float32),
        pltpu.VMEM((8 * 1024,), jnp.float32),
        pltpu.VMEM((8 * 1024,), jnp.float32),
        pltpu.SemaphoreType.DMA,
        pltpu.SemaphoreType.DMA,
        pltpu.SemaphoreType.DMA,
        pltpu.SemaphoreType.DMA,
    ],
    compiler_params=pltpu.CompilerParams(
        use_tc_tiling_on_sc=False, needs_layout_passes=False),
)
def _embed_kernel(table_hbm, idx_hbm, out_hbm, idx_v, jidx0, jidx1,
                  rows0, rows1, slab0, slab1, gsem0, gsem1, osem0, osem1):
    wid = lax.axis_index("s") * _NC + lax.axis_index("c")
    base = wid * _B_PER_W
    jidx = (jidx0, jidx1)
    rows = (rows0, rows1)
    slab = (slab0, slab1)
    gsem = (gsem0, gsem1)
    osem = (osem0, osem1)

    pltpu.sync_copy(idx_hbm.at[pl.ds(base, _B_PER_W)], idx_v)

    iota = lax.iota(jnp.int32, 16)
    iota200 = iota * _NJ
    rot = [lax.rem(iota + r, 16) for r in range(16)]
    ci = [iota + 16 * q for q in range(4)]
    fw = [(iota // 8) * 1024 + (iota % 8) * 128 + r_v for r_v in rot]

    def build_jidx(j, b):
        for grp in range(8):
            pos = iota200 + (grp * 16 * _NJ + j)
            vals = plsc.load_gather(idx_v, [pos])
            jidx[b][pl.ds(grp * 16, 16)] = vals

    def start_gather(b):
        pltpu.async_copy(table_hbm.at[jidx[b]], rows[b], gsem[b])

    def wait_gather(b):
        pltpu.make_async_copy(table_hbm.at[jidx[b]], rows[b], gsem[b]).wait()

    def transpose(b, fast=False):
        def work(g):
            ii0 = g * 16
            for q in range(4):
                wadd = q * 2048 + ii0
                for r in range(16):
                    vals = plsc.load_gather(rows[b], [rot[r] + ii0, ci[q]])
                    plsc.store_scatter(slab[b], [fw[r] + wadd], vals * _SCALE)
        if fast:
            plsc.parallel_loop(0, 8)(work)
        else:
            lax.fori_loop(0, 8, lambda g, c: (work(g), c)[1], 0)

    def start_out(j, b):
        for tk in range(8):
            pltpu.async_copy(
                slab[b].at[pl.ds(tk * 1024, 1024)],
                out_hbm.at[j, tk, wid], osem[b])

    def wait_out(j, b):
        for tk in range(8):
            pltpu.make_async_copy(
                slab[b].at[pl.ds(tk * 1024, 1024)],
                out_hbm.at[j, tk, wid], osem[b]).wait()

    build_jidx(0, 0)
    start_gather(0)
    build_jidx(1, 1)
    start_gather(1)

    for j, b in ((0, 0), (1, 1)):
        wait_gather(b)
        transpose(b)
        build_jidx(j + 2, b)
        start_gather(b)
        start_out(j, b)

    def pair(k, carry):
        j = 2 * k
        for b in (0, 1):
            wait_gather(b)
            wait_out(j + b - 2, b)
            transpose(b, fast=True)
            build_jidx(j + b + 2, b)
            start_gather(b)
            start_out(j + b, b)
        return carry

    lax.fori_loop(1, _NJ // 2 - 1, pair, 0)

    for j, b in ((_NJ - 2, 0), (_NJ - 1, 1)):
        wait_gather(b)
        wait_out(j - 2, b)
        transpose(b)
        start_out(j, b)
    wait_out(_NJ - 2, 0)
    wait_out(_NJ - 1, 1)


def kernel(x, lut):
    idx = x.reshape(-1).astype(jnp.int32)
    y = _embed_kernel(lut, idx)
    return (y.reshape(_NJ, 8, _NW, 8, 128)
             .transpose(2, 4, 0, 1, 3)
             .reshape(_NI, _NJ, _D))


# speedup vs baseline: 2.0788x; 2.0788x over previous
import functools
import math

import jax
import jax.numpy as jnp
from jax import lax
from jax.experimental import pallas as pl
from jax.experimental.pallas import tpu as pltpu
from jax.experimental.pallas import tpu_sc as plsc

_D_MODEL = 100000
_D = 64
_SCALE = math.sqrt(_D_MODEL)
_NC, _NS = 2, 16
_NW = _NC * _NS
_NI, _NJ = 4096, 200
_B = _NI * _NJ
_IB = _NI // _NW
_B_PER_W = _IB * _NJ

_mesh = plsc.VectorSubcoreMesh(core_axis_name="c", subcore_axis_name="s")


@functools.partial(
    pl.kernel,
    out_type=jax.ShapeDtypeStruct((_NJ, 8, _NW, 1024), jnp.float32),
    mesh=_mesh,
    scratch_types=[
        pltpu.VMEM((_B_PER_W,), jnp.int32),
        pltpu.VMEM((_IB,), jnp.int32),
        pltpu.VMEM((_IB,), jnp.int32),
        pltpu.VMEM((_IB, _D), jnp.float32),
        pltpu.VMEM((_IB, _D), jnp.float32),
        pltpu.VMEM((8 * 1024,), jnp.float32),
        pltpu.VMEM((8 * 1024,), jnp.float32),
        pltpu.SemaphoreType.DMA,
        pltpu.SemaphoreType.DMA,
        pltpu.SemaphoreType.DMA,
        pltpu.SemaphoreType.DMA,
    ],
    compiler_params=pltpu.CompilerParams(
        use_tc_tiling_on_sc=False, needs_layout_passes=False),
)
def _embed_kernel(table_hbm, idx_hbm, out_hbm, idx_v, jidx0, jidx1,
                  rows0, rows1, slab0, slab1, gsem0, gsem1, osem0, osem1):
    wid = lax.axis_index("s") * _NC + lax.axis_index("c")
    base = wid * _B_PER_W
    jidx = (jidx0, jidx1)
    rows = (rows0, rows1)
    slab = (slab0, slab1)
    gsem = (gsem0, gsem1)
    osem = (osem0, osem1)

    pltpu.sync_copy(idx_hbm.at[pl.ds(base, _B_PER_W)], idx_v)

    iota = lax.iota(jnp.int32, 16)
    iota200 = iota * _NJ
    rot = [lax.rem(iota + r, 16) for r in range(16)]
    ci = [iota + 16 * q for q in range(4)]
    fw = [(iota // 8) * 1024 + (iota % 8) * 128 + r_v for r_v in rot]

    def build_jidx(j, b):
        for grp in range(8):
            pos = iota200 + (grp * 16 * _NJ + j)
            vals = plsc.load_gather(idx_v, [pos])
            jidx[b][pl.ds(grp * 16, 16)] = vals

    def start_gather(b):
        pltpu.async_copy(table_hbm.at[jidx[b]], rows[b], gsem[b])

    def wait_gather(b):
        pltpu.make_async_copy(table_hbm.at[jidx[b]], rows[b], gsem[b]).wait()

    def transpose(b, fast=False):
        def work(g):
            ii0 = g * 16
            for q in range(4):
                wadd = q * 2048 + ii0
                vals = [
                    plsc.load_gather(rows[b], [rot[r] + ii0, ci[q]]) * _SCALE
                    for r in range(16)
                ]
                for r in range(16):
                    plsc.store_scatter(slab[b], [fw[r] + wadd], vals[r])
        if fast:
            plsc.parallel_loop(0, 8)(work)
        else:
            lax.fori_loop(0, 8, lambda g, c: (work(g), c)[1], 0)

    def start_out(j, b):
        for tk in range(8):
            pltpu.async_copy(
                slab[b].at[pl.ds(tk * 1024, 1024)],
                out_hbm.at[j, tk, wid], osem[b])

    def wait_out(j, b):
        for tk in range(8):
            pltpu.make_async_copy(
                slab[b].at[pl.ds(tk * 1024, 1024)],
                out_hbm.at[j, tk, wid], osem[b]).wait()

    build_jidx(0, 0)
    start_gather(0)
    build_jidx(1, 1)
    start_gather(1)

    for j, b in ((0, 0), (1, 1)):
        wait_gather(b)
        transpose(b)
        build_jidx(j + 2, b)
        start_gather(b)
        start_out(j, b)

    def pair(k, carry):
        j = 2 * k
        for b in (0, 1):
            wait_gather(b)
            wait_out(j + b - 2, b)
            transpose(b, fast=True)
            build_jidx(j + b + 2, b)
            start_gather(b)
            start_out(j + b, b)
        return carry

    lax.fori_loop(1, _NJ // 2 - 1, pair, 0)

    for j, b in ((_NJ - 2, 0), (_NJ - 1, 1)):
        wait_gather(b)
        wait_out(j - 2, b)
        transpose(b)
        start_out(j, b)
    wait_out(_NJ - 2, 0)
    wait_out(_NJ - 1, 1)


def kernel(x, lut):
    idx = x.reshape(-1).astype(jnp.int32)
    y = _embed_kernel(lut, idx)
    return (y.reshape(_NJ, 8, _NW, 8, 128)
             .transpose(2, 4, 0, 1, 3)
             .reshape(_NI, _NJ, _D))


# speedup vs baseline: 2.1572x; 1.0377x over previous
import functools
import math

import jax
import jax.numpy as jnp
from jax import lax
from jax.experimental import pallas as pl
from jax.experimental.pallas import tpu as pltpu
from jax.experimental.pallas import tpu_sc as plsc

_D_MODEL = 100000
_D = 64
_SCALE = math.sqrt(_D_MODEL)
_NC, _NS = 2, 16
_NW = _NC * _NS
_NI, _NJ = 4096, 200
_B = _NI * _NJ
_IB = _NI // _NW
_B_PER_W = _IB * _NJ

_mesh = plsc.VectorSubcoreMesh(core_axis_name="c", subcore_axis_name="s")


@functools.partial(
    pl.kernel,
    out_type=jax.ShapeDtypeStruct((_NJ, 8, _NW, 1024), jnp.float32),
    mesh=_mesh,
    scratch_types=[
        pltpu.VMEM((_NJ // 8, 1, 8, _IB), jnp.int32),
        pltpu.VMEM((_IB, _D), jnp.float32),
        pltpu.VMEM((_IB, _D), jnp.float32),
        pltpu.VMEM((8 * 1024,), jnp.float32),
        pltpu.VMEM((8 * 1024,), jnp.float32),
        pltpu.SemaphoreType.DMA,
        pltpu.SemaphoreType.DMA,
        pltpu.SemaphoreType.DMA,
        pltpu.SemaphoreType.DMA,
    ],
    compiler_params=pltpu.CompilerParams(
        use_tc_tiling_on_sc=False, needs_layout_passes=False),
)
def _embed_kernel(table_hbm, idx_hbm, out_hbm, idx_v,
                  rows0, rows1, slab0, slab1, gsem0, gsem1, osem0, osem1):
    wid = lax.axis_index("s") * _NC + lax.axis_index("c")
    rows = (rows0, rows1)
    slab = (slab0, slab1)
    gsem = (gsem0, gsem1)
    osem = (osem0, osem1)

    pltpu.sync_copy(idx_hbm.at[:, pl.ds(wid, 1), :, :], idx_v)

    iota = lax.iota(jnp.int32, 16)
    rot = [lax.rem(iota + r, 16) for r in range(16)]
    ci = [iota + 16 * q for q in range(4)]
    fw = [(iota // 8) * 1024 + (iota % 8) * 128 + r_v for r_v in rot]

    def jcol(j):
        return idx_v.at[j // 8, 0, lax.rem(j, 8)]

    def start_gather(j, b):
        pltpu.async_copy(table_hbm.at[jcol(j)], rows[b], gsem[b])

    def wait_gather(j, b):
        pltpu.make_async_copy(table_hbm.at[jcol(j)], rows[b], gsem[b]).wait()

    def transpose(b, fast=False):
        def work(g):
            ii0 = g * 16
            for q in range(4):
                wadd = q * 2048 + ii0
                vals = [
                    plsc.load_gather(rows[b], [rot[r] + ii0, ci[q]]) * _SCALE
                    for r in range(16)
                ]
                for r in range(16):
                    plsc.store_scatter(slab[b], [fw[r] + wadd], vals[r])
        if fast:
            plsc.parallel_loop(0, 8)(work)
        else:
            lax.fori_loop(0, 8, lambda g, c: (work(g), c)[1], 0)

    def start_out(j, b):
        for tk in range(8):
            pltpu.async_copy(
                slab[b].at[pl.ds(tk * 1024, 1024)],
                out_hbm.at[j, tk, wid], osem[b])

    def wait_out(j, b):
        for tk in range(8):
            pltpu.make_async_copy(
                slab[b].at[pl.ds(tk * 1024, 1024)],
                out_hbm.at[j, tk, wid], osem[b]).wait()

    start_gather(0, 0)
    start_gather(1, 1)

    for j, b in ((0, 0), (1, 1)):
        wait_gather(j, b)
        transpose(b)
        start_gather(j + 2, b)
        start_out(j, b)

    def pair(k, carry):
        j = 2 * k
        for b in (0, 1):
            wait_gather(j + b, b)
            wait_out(j + b - 2, b)
            transpose(b, fast=True)
            start_gather(j + b + 2, b)
            start_out(j + b, b)
        return carry

    lax.fori_loop(1, _NJ // 2 - 1, pair, 0)

    for j, b in ((_NJ - 2, 0), (_NJ - 1, 1)):
        wait_gather(j, b)
        wait_out(j - 2, b)
        transpose(b)
        start_out(j, b)
    wait_out(_NJ - 2, 0)
    wait_out(_NJ - 1, 1)


def kernel(x, lut):
    idx = (x.astype(jnp.int32).T
           .reshape(_NJ // 8, 8, _NW, _IB)
           .transpose(0, 2, 1, 3))
    y = _embed_kernel(lut, idx)
    return (y.reshape(_NJ, 8, _NW, 8, 128)
             .transpose(2, 4, 0, 1, 3)
             .reshape(_NI, _NJ, _D))


# speedup vs baseline: 2.4904x; 1.1545x over previous
import functools
import math

import jax
import jax.numpy as jnp
from jax import lax
from jax.experimental import pallas as pl
from jax.experimental.pallas import tpu as pltpu
from jax.experimental.pallas import tpu_sc as plsc

_D_MODEL = 100000
_D = 64
_SCALE = math.sqrt(_D_MODEL)
_NC, _NS = 2, 16
_NW = _NC * _NS
_NI, _NJ = 4096, 200
_B = _NI * _NJ
_IB = _NI // _NW
_B_PER_W = _IB * _NJ

_mesh = plsc.VectorSubcoreMesh(core_axis_name="c", subcore_axis_name="s")


@functools.partial(
    pl.kernel,
    out_type=jax.ShapeDtypeStruct((_NJ, 8, _NW, 1024), jnp.float32),
    mesh=_mesh,
    scratch_types=[
        pltpu.VMEM((_NJ // 8, 1, 8, _IB), jnp.int32),
        pltpu.VMEM((_IB, _D), jnp.float32),
        pltpu.VMEM((_IB, _D), jnp.float32),
        pltpu.VMEM((_IB, _D), jnp.float32),
        pltpu.VMEM((8 * 1024,), jnp.float32),
        pltpu.VMEM((8 * 1024,), jnp.float32),
        pltpu.VMEM((8 * 1024,), jnp.float32),
        pltpu.SemaphoreType.DMA,
        pltpu.SemaphoreType.DMA,
        pltpu.SemaphoreType.DMA,
        pltpu.SemaphoreType.DMA,
        pltpu.SemaphoreType.DMA,
        pltpu.SemaphoreType.DMA,
    ],
    compiler_params=pltpu.CompilerParams(
        use_tc_tiling_on_sc=False, needs_layout_passes=False),
)
def _embed_kernel(table_hbm, idx_hbm, out_hbm, idx_v,
                  rows0, rows1, rows2, slab0, slab1, slab2,
                  gsem0, gsem1, gsem2, osem0, osem1, osem2):
    wid = lax.axis_index("s") * _NC + lax.axis_index("c")
    rows = (rows0, rows1, rows2)
    slab = (slab0, slab1, slab2)
    gsem = (gsem0, gsem1, gsem2)
    osem = (osem0, osem1, osem2)

    pltpu.sync_copy(idx_hbm.at[:, pl.ds(wid, 1), :, :], idx_v)

    iota = lax.iota(jnp.int32, 16)
    rot = [lax.rem(iota + r, 16) for r in range(16)]
    ci = [iota + 16 * q for q in range(4)]
    fw = [(iota // 8) * 1024 + (iota % 8) * 128 + r_v for r_v in rot]

    def jcol(j):
        return idx_v.at[j // 8, 0, lax.rem(j, 8)]

    def start_gather(j, b):
        pltpu.async_copy(table_hbm.at[jcol(j)], rows[b], gsem[b])

    def wait_gather(j, b):
        pltpu.make_async_copy(table_hbm.at[jcol(j)], rows[b], gsem[b]).wait()

    def transpose(b, fast=False):
        def work(g):
            ii0 = g * 16
            for q in range(4):
                wadd = q * 2048 + ii0
                vals = [
                    plsc.load_gather(rows[b], [rot[r] + ii0, ci[q]]) * _SCALE
                    for r in range(16)
                ]
                for r in range(16):
                    plsc.store_scatter(slab[b], [fw[r] + wadd], vals[r])
        if fast:
            plsc.parallel_loop(0, 8)(work)
        else:
            lax.fori_loop(0, 8, lambda g, c: (work(g), c)[1], 0)

    def start_out(j, b):
        for tk in range(8):
            pltpu.async_copy(
                slab[b].at[pl.ds(tk * 1024, 1024)],
                out_hbm.at[j, tk, wid], osem[b])

    def wait_out(j, b):
        for tk in range(8):
            pltpu.make_async_copy(
                slab[b].at[pl.ds(tk * 1024, 1024)],
                out_hbm.at[j, tk, wid], osem[b]).wait()

    for b in (0, 1, 2):
        start_gather(b, b)
    for j in (0, 1, 2):
        b = j
        wait_gather(j, b)
        transpose(b)
        start_gather(j + 3, b)
        start_out(j, b)

    def triple(k, carry):
        j0 = 3 * k
        for b in (0, 1, 2):
            j = j0 + b
            wait_gather(j, b)
            wait_out(j - 3, b)
            transpose(b, fast=True)
            start_gather(j + 3, b)
            start_out(j, b)
        return carry

    lax.fori_loop(1, (_NJ - 5) // 3, triple, 0)

    for j in range(_NJ - 5, _NJ):
        b = j % 3
        wait_gather(j, b)
        wait_out(j - 3, b)
        transpose(b)
        if j + 3 < _NJ:
            start_gather(j + 3, b)
        start_out(j, b)
    for j in range(_NJ - 3, _NJ):
        wait_out(j, j % 3)


def kernel(x, lut):
    idx = (x.astype(jnp.int32).T
           .reshape(_NJ // 8, 8, _NW, _IB)
           .transpose(0, 2, 1, 3))
    y = _embed_kernel(lut, idx)
    return (y.reshape(_NJ, 8, _NW, 8, 128)
             .transpose(2, 4, 0, 1, 3)
             .reshape(_NI, _NJ, _D))
